# Initial kernel scaffold; baseline (speedup 1.0000x reference)
import functools

import jax
import jax.numpy as jnp
from jax import lax
from jax.experimental import pallas as pl
from jax.experimental.pallas import tpu as pltpu
from jax.experimental.pallas import tpu_sc as plsc

V = 100000
F = 4
D = 64
B = 16384

NC, NS, L = 2, 16, 16
NW = NC * NS
BW = B // NW
CHUNK = 128
NCH = BW // CHUNK

_mesh = plsc.VectorSubcoreMesh(core_axis_name="c", subcore_axis_name="s")


@functools.partial(
    pl.kernel,
    out_type=jax.ShapeDtypeStruct((B, D), jnp.float32),
    mesh=_mesh,
    scratch_types=[
        pltpu.VMEM((F, NCH, CHUNK), jnp.int32),
        pltpu.VMEM((CHUNK, D), jnp.float32),
        pltpu.VMEM((CHUNK, D), jnp.float32),
        pltpu.VMEM((CHUNK, D), jnp.float32),
        pltpu.VMEM((CHUNK, D), jnp.float32),
        pltpu.VMEM((CHUNK, F), jnp.float32),
        pltpu.VMEM((F, CHUNK), jnp.float32),
        pltpu.VMEM((CHUNK, D), jnp.float32),
        pltpu.SemaphoreType.DMA,
    ],
)
def _eges_sc(idx_hbm, tab_hbm, alpha_hbm, out_hbm,
             idx_v, r0, r1, r2, r3, a_buf, scales, out_buf, sem):
    wid = lax.axis_index("s") * NC + lax.axis_index("c")
    base = wid * BW

    for f in range(F):
        pltpu.sync_copy(idx_hbm.at[f, wid], idx_v.at[f])

    rs = (r0, r1, r2, r3)

    def chunk_body(c, carry):
        cps = [pltpu.async_copy(tab_hbm.at[idx_v.at[f, c]], rs[f], sem)
               for f in range(F)]
        cps.append(pltpu.async_copy(alpha_hbm.at[idx_v.at[0, c]], a_buf, sem))
        for cp in cps:
            cp.wait()

        iota = lax.iota(jnp.int32, (L,))
        for g in range(CHUNK // L):
            row_idx = g * L + iota
            a = [plsc.load_gather(a_buf,
                                  [row_idx, jnp.full((L,), f, jnp.int32)])
                 for f in range(F)]
            w = [jnp.exp(x) for x in a]
            u = [jnp.exp(x) for x in w]
            denom = (u[0] + u[1]) + (u[2] + u[3])
            for f in range(F):
                scales[f, pl.ds(g * L, L)] = w[f] / denom

        def row_body(i, carry2):
            s0 = scales[0, i]
            s1 = scales[1, i]
            s2 = scales[2, i]
            s3 = scales[3, i]
            for j in range(D // L):
                sl = pl.ds(j * L, L)
                out_buf[i, sl] = (r0[i, sl] * s0 + r1[i, sl] * s1
                                  + r2[i, sl] * s2 + r3[i, sl] * s3)
            return carry2

        lax.fori_loop(0, CHUNK, row_body, 0)
        pltpu.sync_copy(out_buf, out_hbm.at[pl.ds(base + c * CHUNK, CHUNK)])
        return carry

    lax.fori_loop(0, NCH, chunk_body, 0)


def kernel(inputs, tables, alpha):
    idx = inputs.astype(jnp.int32) + (jnp.arange(F, dtype=jnp.int32) * V)[None, :]
    idx_t = idx.T.reshape(F, NW, NCH, CHUNK)
    tab2d = tables.reshape(F * V, D)
    return _eges_sc(idx_t, tab2d, alpha)


# speedup vs baseline: 1.1823x; 1.1823x over previous
import functools

import jax
import jax.numpy as jnp
from jax import lax
from jax.experimental import pallas as pl
from jax.experimental.pallas import tpu as pltpu
from jax.experimental.pallas import tpu_sc as plsc

V = 100000
F = 4
D = 64
B = 16384

NC, NS, L = 2, 16, 16
NW = NC * NS
BW = B // NW
CHUNK = 128
NCH = BW // CHUNK

_mesh = plsc.VectorSubcoreMesh(core_axis_name="c", subcore_axis_name="s")


@functools.partial(
    pl.kernel,
    out_type=jax.ShapeDtypeStruct((B, D), jnp.float32),
    mesh=_mesh,
    scratch_types=[
        pltpu.VMEM((F, NCH, CHUNK), jnp.int32),
        pltpu.VMEM((F, NCH, CHUNK), jnp.int32),
        pltpu.VMEM((CHUNK, D), jnp.float32),
        pltpu.VMEM((CHUNK, D), jnp.float32),
        pltpu.VMEM((CHUNK, D), jnp.float32),
        pltpu.VMEM((CHUNK, D), jnp.float32),
        pltpu.VMEM((F, CHUNK), jnp.float32),
        pltpu.VMEM((F * CHUNK,), jnp.float32),
        pltpu.VMEM((CHUNK, D), jnp.float32),
        pltpu.SemaphoreType.DMA,
    ],
    compiler_params=pltpu.CompilerParams(needs_layout_passes=False,
                                         use_tc_tiling_on_sc=False),
)
def _eges_sc(idx_hbm, aidx_hbm, tab_hbm, alphat_hbm, out_hbm,
             idx_v, aidx_v, r0, r1, r2, r3, a_buf, scales, out_buf, sem):
    wid = lax.axis_index("s") * NC + lax.axis_index("c")
    base = wid * BW

    for f in range(F):
        pltpu.sync_copy(idx_hbm.at[f, wid], idx_v.at[f])
        pltpu.sync_copy(aidx_hbm.at[f, wid], aidx_v.at[f])

    rs = (r0, r1, r2, r3)

    def chunk_body(c, carry):
        cps = [pltpu.async_copy(tab_hbm.at[idx_v.at[f, c]], rs[f], sem)
               for f in range(F)]
        cps += [pltpu.async_copy(alphat_hbm.at[aidx_v.at[f, c]],
                                 a_buf.at[f], sem)
                for f in range(F)]
        for cp in cps:
            cp.wait()

        for g in range(CHUNK // L):
            sl = pl.ds(g * L, L)
            a = [a_buf[f, sl] for f in range(F)]
            w = [jnp.exp(x) for x in a]
            u = [jnp.exp(x) for x in w]
            denom = (u[0] + u[1]) + (u[2] + u[3])
            for f in range(F):
                scales[pl.ds(f * CHUNK + g * L, L)] = w[f] / denom

        def row_body(i, carry2):
            col = jnp.full((L,), i, jnp.int32)
            s0 = plsc.load_gather(scales, [col])
            s1 = plsc.load_gather(scales, [CHUNK + col])
            s2 = plsc.load_gather(scales, [2 * CHUNK + col])
            s3 = plsc.load_gather(scales, [3 * CHUNK + col])
            for j in range(D // L):
                sl = pl.ds(j * L, L)
                out_buf[i, sl] = (r0[i, sl] * s0 + r1[i, sl] * s1
                                  + r2[i, sl] * s2 + r3[i, sl] * s3)
            return carry2

        lax.fori_loop(0, CHUNK, row_body, 0, unroll=4)
        pltpu.sync_copy(out_buf, out_hbm.at[pl.ds(base + c * CHUNK, CHUNK)])
        return carry

    lax.fori_loop(0, NCH, chunk_body, 0)


def kernel(inputs, tables, alpha):
    inputs = inputs.astype(jnp.int32)
    foffs = (jnp.arange(F, dtype=jnp.int32) * V)[None, :]
    idx_t = (inputs + foffs).T.reshape(F, NW, NCH, CHUNK)
    aidx_t = (inputs[:, 0:1] + foffs).T.reshape(F, NW, NCH, CHUNK)
    tab2d = tables.reshape(F * V, D)
    alphat = alpha.T.reshape(F * V)
    return _eges_sc(idx_t, aidx_t, tab2d, alphat)
